# Initial kernel scaffold; baseline (speedup 1.0000x reference)
import jax
import jax.numpy as jnp
from jax.experimental import pallas as pl


def kernel(token_ids, tid2eid):
    raise NotImplementedError("write your pallas kernel here")


# speedup vs baseline: 7.4863x; 7.4863x over previous
import functools

import jax
import jax.numpy as jnp
from jax import lax
from jax.experimental import pallas as pl
from jax.experimental.pallas import tpu as pltpu
from jax.experimental.pallas import tpu_sc as plsc

NUM_EXPERTS = 64
TOPK = 2
_NW = 32
_LANES = 16


@functools.cache
def _build_router(n):
    assert n % (_NW * 128) == 0
    tpw = n // _NW
    rows = tpw // 128
    groups = tpw // _LANES

    mesh = plsc.VectorSubcoreMesh(core_axis_name="c", subcore_axis_name="s")

    @functools.partial(
        pl.kernel,
        mesh=mesh,
        compiler_params=pltpu.CompilerParams(needs_layout_passes=False),
        out_type=[
            jax.ShapeDtypeStruct((n * NUM_EXPERTS,), jnp.float32),
            jax.ShapeDtypeStruct((n * (NUM_EXPERTS // 4),), jnp.int32),
        ],
        scratch_types=[
            pltpu.VMEM((rows, 128), jnp.int32),
            pltpu.VMEM((tpw,), jnp.int32),
            pltpu.VMEM((tpw,), jnp.int32),
            pltpu.VMEM((tpw * NUM_EXPERTS,), jnp.float32),
            pltpu.VMEM((tpw * (NUM_EXPERTS // 4),), jnp.int32),
            pltpu.SemaphoreType.DMA,
        ],
    )
    def router(ids_hbm, t2e0_hbm, t2e1_hbm, probs_hbm, mapw_hbm,
               ids_v, e0_v, e1_v, probs_v, mapw_v, sem):
        c = lax.axis_index("c")
        s = lax.axis_index("s")
        wid = s * 2 + c

        pltpu.sync_copy(ids_hbm.at[pl.ds(wid * rows, rows)], ids_v)

        copies = []
        for j in range(rows):
            copies.append(pltpu.async_copy(
                t2e0_hbm.at[ids_v.at[j]], e0_v.at[pl.ds(j * 128, 128)], sem))
            copies.append(pltpu.async_copy(
                t2e1_hbm.at[ids_v.at[j]], e1_v.at[pl.ds(j * 128, 128)], sem))

        zf = jnp.zeros((_LANES,), jnp.float32)
        zi = jnp.zeros((_LANES,), jnp.int32)

        def zero_probs(i, carry):
            for k in range(8):
                probs_v[pl.ds(i * 128 + k * 16, 16)] = zf
            return carry
        lax.fori_loop(0, tpw * NUM_EXPERTS // 128, zero_probs, 0)

        def zero_map(i, carry):
            for k in range(8):
                mapw_v[pl.ds(i * 128 + k * 16, 16)] = zi
            return carry
        lax.fori_loop(0, tpw * (NUM_EXPERTS // 4) // 128, zero_map, 0)

        for cp in copies:
            cp.wait()

        iota = lax.iota(jnp.int32, _LANES)
        half = jnp.full((_LANES,), 0.5, jnp.float32)
        one = jnp.full((_LANES,), 1, jnp.int32)

        def scatter(g, carry):
            t = g * _LANES + iota
            pos = t * NUM_EXPERTS
            wbase = t * (NUM_EXPERTS // 4)
            for ev in (e0_v, e1_v):
                e = ev[pl.ds(g * _LANES, _LANES)]
                plsc.store_scatter(probs_v, [pos + e], half)
                w = wbase + lax.shift_right_logical(e, 2)
                byte = lax.shift_left(one, lax.shift_left(e & 3, 3))
                plsc.addupdate_scatter(mapw_v, [w], byte)
            return carry
        lax.fori_loop(0, groups, scatter, 0)

        pltpu.sync_copy(
            probs_v,
            probs_hbm.at[pl.ds(wid * tpw * NUM_EXPERTS, tpw * NUM_EXPERTS)])
        pltpu.sync_copy(
            mapw_v,
            mapw_hbm.at[pl.ds(wid * tpw * (NUM_EXPERTS // 4),
                              tpw * (NUM_EXPERTS // 4))])

    return router


def kernel(token_ids, tid2eid):
    b, s = token_ids.shape
    n = b * s
    ids2d = token_ids.reshape(n // 128, 128)
    t2e0 = tid2eid[:, 0]
    t2e1 = tid2eid[:, 1]
    probs_flat, mapw = _build_router(n)(ids2d, t2e0, t2e1)
    probs = probs_flat.reshape(n, NUM_EXPERTS)
    map_u8 = jax.lax.bitcast_convert_type(mapw, jnp.uint8)
    routing_map = map_u8.reshape(n, NUM_EXPERTS) != 0
    return probs, routing_map
